# baseline (speedup 1.0000x reference)
import functools

import jax
import jax.numpy as jnp
from jax import lax
from jax.experimental import pallas as pl
from jax.experimental.pallas import tpu as pltpu
from jax.experimental.pallas import tpu_sc as plsc

N = 10000
D = 128
B = 256
NC = 2
NS = 16
NW = NC * NS
SEG = B // NW
NCHUNK = N // 16
XR = 384
CW = 16
NLANE = 16


def _lane0(v):
    return lax.squeeze(lax.slice(v, (0,), (1,)), (0,))


def _sc_segment_sum(x, batch_i32):
    mesh = plsc.VectorSubcoreMesh(core_axis_name="c", subcore_axis_name="s")

    @functools.partial(
        pl.kernel,
        out_type=(
            jax.ShapeDtypeStruct((B * D,), jnp.float32),
            jax.ShapeDtypeStruct((B * CW,), jnp.float32),
        ),
        mesh=mesh,
        scratch_types=[
            pltpu.VMEM((XR, D), jnp.float32),
            pltpu.VMEM((N,), jnp.int32),
            pltpu.VMEM(((SEG + 1) * D,), jnp.float32),
            pltpu.VMEM(((SEG + 1) * CW,), jnp.float32),
        ],
    )
    def seg_sum(x_hbm, batch_hbm, sums_hbm, cnts_hbm,
                xrows, ids, lacc, lcnt):
        c = lax.axis_index("c")
        s = lax.axis_index("s")
        wid = s * NC + c
        t0 = wid * SEG

        pltpu.sync_copy(batch_hbm, ids)

        zero16 = jnp.zeros((NLANE,), jnp.float32)
        for r in range(SEG + 1):
            for k in range(D // NLANE):
                lacc[pl.ds(r * D + k * NLANE, NLANE)] = zero16
            lcnt[pl.ds(r * CW, NLANE)] = zero16

        def count_chunks_below(t):
            def step(_, lohi):
                lo, hi = lohi
                mid = (lo + hi) // 2
                v = _lane0(ids[pl.ds(pl.multiple_of(mid * 16, 16), NLANE)])
                pred = v < t
                return (jnp.where(pred, mid + 1, lo), jnp.where(pred, hi, mid))
            lo, _ = lax.fori_loop(0, 10, step, (jnp.int32(0),
                                                jnp.int32(NCHUNK)))
            return lo

        cl0 = count_chunks_below(t0)
        cl1 = count_chunks_below(t0 + SEG)
        start_al = jnp.maximum(cl0 - 1, 0) * 16
        end_al = jnp.minimum(cl1 + 1, NCHUNK) * 16
        n_rows = end_al - start_al
        npass = (n_rows + XR - 1) // XR

        def do_pass(p, carry):
            accs, cnt, prev = carry
            p0 = start_al + p * XR
            r0 = pl.multiple_of(jnp.minimum(p0, N - XR), 16)
            pltpu.sync_copy(x_hbm.at[pl.ds(r0, XR)], xrows)
            ngrp = jnp.minimum(XR, end_al - p0) // 16

            def do_group(g, carry_g):
                accs_g, cnt_g, prev_g = carry_g
                grow = pl.multiple_of(p0 + g * 16, 16)
                sidv = ids[pl.ds(grow, NLANE)]
                loc = grow - r0
                for j in range(NLANE):
                    sid = lax.squeeze(lax.slice(sidv, (j,), (j + 1,)), (0,))
                    same = sid == prev_g
                    row = loc + j
                    new_accs = []
                    for k in range(D // NLANE):
                        xk = xrows[row, pl.ds(k * NLANE, NLANE)]
                        new_accs.append(jnp.where(same, accs_g[k] + xk, xk))
                    accs_g = new_accs
                    cnt_g = jnp.where(same, cnt_g + 1.0,
                                      jnp.ones((NLANE,), jnp.float32))
                    sl = sid - t0
                    ok = jnp.logical_and(sl >= 0, sl < SEG)
                    rs = jnp.where(ok, sl, SEG)
                    for k in range(D // NLANE):
                        lacc[pl.ds(rs * D + k * NLANE, NLANE)] = accs_g[k]
                    lcnt[pl.ds(rs * CW, NLANE)] = cnt_g
                    prev_g = sid
                return (accs_g, cnt_g, prev_g)

            return lax.fori_loop(0, ngrp, do_group, (accs, cnt, prev))

        init = ([jnp.zeros((NLANE,), jnp.float32) for _ in range(D // NLANE)],
                jnp.zeros((NLANE,), jnp.float32), jnp.int32(-1))
        lax.fori_loop(0, npass, do_pass, init)

        pltpu.sync_copy(lacc.at[pl.ds(0, SEG * D)],
                        sums_hbm.at[pl.ds(pl.multiple_of(t0 * D, D),
                                          SEG * D)])
        pltpu.sync_copy(lcnt.at[pl.ds(0, SEG * CW)],
                        cnts_hbm.at[pl.ds(pl.multiple_of(t0 * CW, CW),
                                          SEG * CW)])

    sums_flat, cnts_flat = seg_sum(x, batch_i32)
    return sums_flat.reshape(B, D), cnts_flat.reshape(B, CW)


def _mlp(sums, cnts, u, w1u, w1a, b1, w2, b2):
    def body(sums_ref, cnts_ref, u_ref, w1u_ref, w1a_ref, b1_ref, w2_ref,
             b2_ref, o_ref):
        cnt = cnts_ref[...]
        inv = 1.0 / jnp.maximum(cnt[:, 0:1], 1.0)
        agg = sums_ref[...] * inv
        h = jnp.dot(u_ref[...], w1u_ref[...], preferred_element_type=jnp.float32)
        h = h + jnp.dot(agg, w1a_ref[...], preferred_element_type=jnp.float32)
        h = jnp.maximum(h + b1_ref[...], 0.0)
        o_ref[...] = (jnp.dot(h, w2_ref[...], preferred_element_type=jnp.float32)
                      + b2_ref[...])

    return pl.pallas_call(
        body,
        out_shape=jax.ShapeDtypeStruct((B, w2.shape[1]), jnp.float32),
    )(sums, cnts, u, w1u, w1a, b1.reshape(1, -1), w2, b2.reshape(1, -1))


def kernel(x, edge_index, edge_attr, u, batch, W1, b1, W2, b2):
    del edge_index, edge_attr
    batch32 = batch.astype(jnp.int32)
    sums, cnts = _sc_segment_sum(x, batch32)
    return _mlp(sums, cnts, u, W1[:D], W1[D:], b1, W2, b2)


# speedup vs baseline: 1.0028x; 1.0028x over previous
import functools

import jax
import jax.numpy as jnp
from jax import lax
from jax.experimental import pallas as pl
from jax.experimental.pallas import tpu as pltpu
from jax.experimental.pallas import tpu_sc as plsc

N = 10000
D = 128
B = 256
NC = 2
NS = 16
NW = NC * NS
SEG = B // NW
NCHUNK = N // 16
XR = 384
CW = 16
NLANE = 16


def _lane0(v):
    return lax.squeeze(lax.slice(v, (0,), (1,)), (0,))


def _sc_segment_sum(x, batch_i32):
    mesh = plsc.VectorSubcoreMesh(core_axis_name="c", subcore_axis_name="s")

    @functools.partial(
        pl.kernel,
        out_type=(
            jax.ShapeDtypeStruct((B * D,), jnp.float32),
            jax.ShapeDtypeStruct((B * CW,), jnp.float32),
        ),
        mesh=mesh,
        scratch_types=[
            pltpu.VMEM((XR, D), jnp.float32),
            pltpu.VMEM((N,), jnp.int32),
            pltpu.VMEM(((SEG + 1) * D,), jnp.float32),
            pltpu.VMEM(((SEG + 1) * CW,), jnp.float32),
        ],
    )
    def seg_sum(x_hbm, batch_hbm, sums_hbm, cnts_hbm,
                xrows, ids, lacc, lcnt):
        c = lax.axis_index("c")
        s = lax.axis_index("s")
        wid = s * NC + c
        t0 = wid * SEG

        pltpu.sync_copy(batch_hbm, ids)

        zero16 = jnp.zeros((NLANE,), jnp.float32)
        for r in range(SEG + 1):
            for k in range(D // NLANE):
                lacc[pl.ds(r * D + k * NLANE, NLANE)] = zero16
            lcnt[pl.ds(r * CW, NLANE)] = zero16

        def count_chunks_below(t):
            def step(_, lohi):
                lo, hi = lohi
                mid = (lo + hi) // 2
                v = _lane0(ids[pl.ds(pl.multiple_of(mid * 16, 16), NLANE)])
                pred = v < t
                return (jnp.where(pred, mid + 1, lo), jnp.where(pred, hi, mid))
            lo, _ = lax.fori_loop(0, 10, step, (jnp.int32(0),
                                                jnp.int32(NCHUNK)))
            return lo

        cl0 = count_chunks_below(t0)
        cl1 = count_chunks_below(t0 + SEG)
        start_al = jnp.maximum(cl0 - 1, 0) * 16
        end_al = jnp.minimum(cl1 + 1, NCHUNK) * 16
        n_rows = end_al - start_al
        npass = (n_rows + XR - 1) // XR

        def do_pass(p, carry):
            accs, cnt, prev = carry
            p0 = start_al + p * XR
            r0 = pl.multiple_of(jnp.minimum(p0, N - XR), 16)
            pltpu.sync_copy(x_hbm.at[pl.ds(r0, XR)], xrows)
            ngrp = jnp.minimum(XR, end_al - p0) // 16

            def do_group(g, carry_g):
                accs_g, cnt_g, prev_g = carry_g
                grow = pl.multiple_of(p0 + g * 16, 16)
                sidv = ids[pl.ds(grow, NLANE)]
                loc = grow - r0
                for j in range(NLANE):
                    sid = lax.squeeze(lax.slice(sidv, (j,), (j + 1,)), (0,))
                    same = sid == prev_g
                    row = loc + j
                    new_accs = []
                    for k in range(D // NLANE):
                        xk = xrows[row, pl.ds(k * NLANE, NLANE)]
                        new_accs.append(jnp.where(same, accs_g[k] + xk, xk))
                    accs_g = new_accs
                    cnt_g = jnp.where(same, cnt_g + 1.0,
                                      jnp.ones((NLANE,), jnp.float32))
                    sl = sid - t0
                    ok = jnp.logical_and(sl >= 0, sl < SEG)
                    rs = jnp.where(ok, sl, SEG)
                    for k in range(D // NLANE):
                        lacc[pl.ds(rs * D + k * NLANE, NLANE)] = accs_g[k]
                    lcnt[pl.ds(rs * CW, NLANE)] = cnt_g
                    prev_g = sid
                return (accs_g, cnt_g, prev_g)

            return lax.fori_loop(0, ngrp, do_group, (accs, cnt, prev))

        init = ([jnp.zeros((NLANE,), jnp.float32) for _ in range(D // NLANE)],
                jnp.zeros((NLANE,), jnp.float32), jnp.int32(-1))
        lax.fori_loop(0, npass, do_pass, init)

        pltpu.sync_copy(lacc.at[pl.ds(0, SEG * D)],
                        sums_hbm.at[pl.ds(pl.multiple_of(t0 * D, D),
                                          SEG * D)])
        pltpu.sync_copy(lcnt.at[pl.ds(0, SEG * CW)],
                        cnts_hbm.at[pl.ds(pl.multiple_of(t0 * CW, CW),
                                          SEG * CW)])

    return seg_sum(x, batch_i32)


def _mlp(sums, cnts, u, w1u, w1a, b1, w2, b2):
    def body(sums_ref, cnts_ref, u_ref, w1u_ref, w1a_ref, b1_ref, w2_ref,
             b2_ref, o_ref):
        cnt = cnts_ref[...]
        inv = 1.0 / jnp.maximum(cnt[:, 0:1], 1.0)
        agg = sums_ref[...].reshape(B, D) * inv
        h = jnp.dot(u_ref[...], w1u_ref[...], preferred_element_type=jnp.float32)
        h = h + jnp.dot(agg, w1a_ref[...], preferred_element_type=jnp.float32)
        h = jnp.maximum(h + b1_ref[...], 0.0)
        o_ref[...] = (jnp.dot(h, w2_ref[...], preferred_element_type=jnp.float32)
                      + b2_ref[...])

    return pl.pallas_call(
        body,
        out_shape=jax.ShapeDtypeStruct((B, w2.shape[1]), jnp.float32),
    )(sums, cnts, u, w1u, w1a, b1.reshape(1, -1), w2, b2.reshape(1, -1))


def kernel(x, edge_index, edge_attr, u, batch, W1, b1, W2, b2):
    del edge_index, edge_attr
    batch32 = batch.astype(jnp.int32)
    sums_flat, cnts_flat = _sc_segment_sum(x, batch32)
    return _mlp(sums_flat, cnts_flat.reshape(B, CW), u, W1[:D], W1[D:],
                b1, W2, b2)
